# baseline (speedup 1.0000x reference)
import functools

import jax
import jax.numpy as jnp
from jax import lax
from jax.experimental import pallas as pl
from jax.experimental.pallas import tpu as pltpu
from jax.experimental.pallas import tpu_sc as plsc


_TS_A = 2048
_TS_B = 2048


def _stage_a_body(x_ref, wt_ref, p1_ref, p2_ref):
    wsum = jnp.sum(wt_ref[...], axis=1, keepdims=True)
    n = wsum.shape[0] // 2
    w1 = wsum[0:n, :].reshape(1, n, 1)
    w2 = wsum[n:, :].reshape(1, n, 1)
    x = x_ref[...]
    p1_ref[...] = jnp.sum(x * w1, axis=1)
    p2_ref[...] = jnp.sum(x * w2, axis=1)


def _stage_a(In, wt):
    B, NSpec, S = In.shape
    nt = pl.cdiv(S, _TS_A)
    return pl.pallas_call(
        _stage_a_body,
        grid=(nt,),
        in_specs=[
            pl.BlockSpec((B, NSpec, _TS_A), lambda i: (0, 0, i)),
            pl.BlockSpec(wt.shape, lambda i: (0, 0)),
        ],
        out_specs=[
            pl.BlockSpec((B, _TS_A), lambda i: (0, i)),
            pl.BlockSpec((B, _TS_A), lambda i: (0, i)),
        ],
        out_shape=[
            jax.ShapeDtypeStruct((B, S), jnp.float32),
            jax.ShapeDtypeStruct((B, S), jnp.float32),
        ],
    )(In, wt)


def _sc_gather(p1, p2, nn):
    B, S = p2.shape
    Z = nn.shape[0]
    info = plsc.get_sparse_core_info()
    nw = info.num_cores * info.num_subcores
    per = (Z * B) // nw
    mesh = plsc.VectorSubcoreMesh(core_axis_name="c", subcore_axis_name="s")

    @functools.partial(
        pl.kernel,
        mesh=mesh,
        out_type=jax.ShapeDtypeStruct((Z, B, S), jnp.float32),
        compiler_params=pltpu.CompilerParams(needs_layout_passes=False),
        scratch_types=[
            pltpu.VMEM((S,), jnp.float32),
            pltpu.VMEM((S,), jnp.float32),
            pltpu.VMEM((2, S), jnp.int32),
            pltpu.VMEM((2, S), jnp.float32),
            pltpu.SemaphoreType.DMA,
            pltpu.SemaphoreType.DMA,
            pltpu.SemaphoreType.DMA,
        ],
    )
    def k(
        p1_hbm, p2_hbm, nn_hbm, g_hbm,
        p1_v, p2_v, idx_v, out_v, p_sem, idx_sem, w_sem,
    ):
        wid = lax.axis_index("s") * info.num_cores + lax.axis_index("c")
        b = wid % B
        zg = wid // B
        d1 = pltpu.async_copy(p1_hbm.at[b], p1_v, p_sem)
        d2 = pltpu.async_copy(p2_hbm.at[b], p2_v, p_sem)
        pltpu.async_copy(nn_hbm.at[zg * per], idx_v.at[0], idx_sem)
        d1.wait()
        d2.wait()
        for j in range(per):
            z = zg * per + j
            buf = j % 2
            pltpu.make_async_copy(
                nn_hbm.at[z], idx_v.at[buf], idx_sem
            ).wait()
            if j >= 2:
                pltpu.make_async_copy(
                    out_v.at[buf], g_hbm.at[zg * per + j - 2, b], w_sem
                ).wait()
            if j + 1 < per:
                pltpu.async_copy(
                    nn_hbm.at[z + 1], idx_v.at[(j + 1) % 2], idx_sem
                )

            @plsc.parallel_loop(0, S, 16, unroll=8)
            def _(i):
                sl = pl.ds(i, 16)
                out_v[buf, sl] = (
                    plsc.load_gather(p2_v, [idx_v[buf, sl]]) + p1_v[sl]
                ) * _LOG2E

            pltpu.async_copy(out_v.at[buf], g_hbm.at[z, b], w_sem)
        for j in range(max(0, per - 2), per):
            pltpu.make_async_copy(
                out_v.at[j % 2], g_hbm.at[zg * per + j, b], w_sem
            ).wait()

    return k(p1, p2, nn)


_LOG2E = 1.4426950408889634
_LN2 = 0.6931471805599453


_CH_B = 256
_ZGRP = 8


def _stage_b_body(g_ref, bt_ref, out_ref):
    bs = jnp.sum(bt_ref[...], axis=1, keepdims=True) * _LOG2E
    B = out_ref.shape[0]
    Z = g_ref.shape[0]
    nch = _TS_B // _CH_B
    lhs = jnp.concatenate(
        [bs, jnp.ones_like(bs)], axis=1
    )
    ones_row = jnp.ones((1, _CH_B), jnp.float32)

    def chunk(c, carry):
        sl = pl.ds(c * _CH_B, _CH_B)
        for b in range(B):
            acc = None
            for z0 in range(0, Z, _ZGRP):
                uprod = None
                for zz in range(z0, z0 + _ZGRP):
                    rhs = jnp.concatenate(
                        [ones_row, g_ref[zz, b, sl][None, :]], axis=0
                    )
                    y = jax.lax.dot_general(
                        lhs,
                        rhs,
                        (((1,), (0,)), ((), ())),
                        preferred_element_type=jnp.float32,
                    )
                    yi = jax.lax.bitcast_convert_type(y, jnp.int32)
                    ny = jax.lax.bitcast_convert_type(
                        yi | jnp.int32(-2147483648), jnp.float32
                    )
                    u = 1.0 + jnp.exp2(ny)
                    uprod = u if uprod is None else uprod * u
                    m = jnp.maximum(y, 0.0)
                    acc = m if acc is None else acc + m
                acc = acc + jnp.log2(uprod)
            out_ref[b, :, sl] = acc * _LN2
        return carry

    lax.fori_loop(0, nch, chunk, 0, unroll=2)


def _stage_b(g, bt):
    Z, B, S = g.shape
    NSpec = bt.shape[0]
    nt = pl.cdiv(S, _TS_B)
    return pl.pallas_call(
        _stage_b_body,
        grid=(nt,),
        in_specs=[
            pl.BlockSpec((Z, B, _TS_B), lambda t: (0, 0, t)),
            pl.BlockSpec(bt.shape, lambda t: (0, 0)),
        ],
        out_specs=pl.BlockSpec((B, NSpec, _TS_B), lambda t: (0, 0, t)),
        out_shape=jax.ShapeDtypeStruct((B, NSpec, S), jnp.float32),
    )(g, bt)


def kernel(In, NNsites, Weights, bias):
    wt = Weights[:, 0, :].T
    bt = bias.T
    nn = NNsites[1:]
    p1, p2 = _stage_a(In, wt)
    g = _sc_gather(p1, p2, nn)
    return _stage_b(g, bt)


# speedup vs baseline: 1.1365x; 1.1365x over previous
import functools

import jax
import jax.numpy as jnp
from jax import lax
from jax.experimental import pallas as pl
from jax.experimental.pallas import tpu as pltpu
from jax.experimental.pallas import tpu_sc as plsc


_TS_A = 2048
_TS_B = 2048


def _stage_a_body(x_ref, wt_ref, p1_ref, p2_ref):
    wsum = jnp.sum(wt_ref[...], axis=1, keepdims=True)
    n = wsum.shape[0] // 2
    w1 = wsum[0:n, :].reshape(1, n, 1)
    w2 = wsum[n:, :].reshape(1, n, 1)
    x = x_ref[...]
    p1_ref[...] = jnp.sum(x * w1, axis=1)
    p2_ref[...] = jnp.sum(x * w2, axis=1)


def _stage_a(In, wt):
    B, NSpec, S = In.shape
    nt = pl.cdiv(S, _TS_A)
    return pl.pallas_call(
        _stage_a_body,
        grid=(nt,),
        in_specs=[
            pl.BlockSpec((B, NSpec, _TS_A), lambda i: (0, 0, i)),
            pl.BlockSpec(wt.shape, lambda i: (0, 0)),
        ],
        out_specs=[
            pl.BlockSpec((B, _TS_A), lambda i: (0, i)),
            pl.BlockSpec((B, _TS_A), lambda i: (0, i)),
        ],
        out_shape=[
            jax.ShapeDtypeStruct((B, S), jnp.float32),
            jax.ShapeDtypeStruct((B, S), jnp.float32),
        ],
    )(In, wt)


def _sc_gather(p1, p2, nn):
    B, S = p2.shape
    Z = nn.shape[0]
    info = plsc.get_sparse_core_info()
    nw = info.num_cores * info.num_subcores
    per = (Z * B) // nw
    mesh = plsc.VectorSubcoreMesh(core_axis_name="c", subcore_axis_name="s")

    @functools.partial(
        pl.kernel,
        mesh=mesh,
        out_type=jax.ShapeDtypeStruct((Z, B, S), jnp.float32),
        compiler_params=pltpu.CompilerParams(needs_layout_passes=False),
        scratch_types=[
            pltpu.VMEM((S,), jnp.float32),
            pltpu.VMEM((S,), jnp.float32),
            pltpu.VMEM((2, S), jnp.int32),
            pltpu.VMEM((2, S), jnp.float32),
            pltpu.SemaphoreType.DMA,
            pltpu.SemaphoreType.DMA,
            pltpu.SemaphoreType.DMA,
        ],
    )
    def k(
        p1_hbm, p2_hbm, nn_hbm, g_hbm,
        p1_v, p2_v, idx_v, out_v, p_sem, idx_sem, w_sem,
    ):
        wid = lax.axis_index("s") * info.num_cores + lax.axis_index("c")
        b = wid % B
        zg = wid // B
        d1 = pltpu.async_copy(p1_hbm.at[b], p1_v, p_sem)
        d2 = pltpu.async_copy(p2_hbm.at[b], p2_v, p_sem)
        pltpu.async_copy(nn_hbm.at[zg * per], idx_v.at[0], idx_sem)
        d1.wait()
        d2.wait()
        for j in range(per):
            z = zg * per + j
            buf = j % 2
            pltpu.make_async_copy(
                nn_hbm.at[z], idx_v.at[buf], idx_sem
            ).wait()
            if j >= 2:
                pltpu.make_async_copy(
                    out_v.at[buf], g_hbm.at[zg * per + j - 2, b], w_sem
                ).wait()
            if j + 1 < per:
                pltpu.async_copy(
                    nn_hbm.at[z + 1], idx_v.at[(j + 1) % 2], idx_sem
                )

            @plsc.parallel_loop(0, S, 16, unroll=8)
            def _(i):
                sl = pl.ds(i, 16)
                out_v[buf, sl] = (
                    plsc.load_gather(p2_v, [idx_v[buf, sl]]) + p1_v[sl]
                ) * _LOG2E

            pltpu.async_copy(out_v.at[buf], g_hbm.at[z, b], w_sem)
        for j in range(max(0, per - 2), per):
            pltpu.make_async_copy(
                out_v.at[j % 2], g_hbm.at[zg * per + j, b], w_sem
            ).wait()

    return k(p1, p2, nn)


_LOG2E = 1.4426950408889634
_LN2 = 0.6931471805599453


_CH_B = 256
_ZGRP = 8


def _stage_b_body(g_ref, bt_ref, out_ref):
    bs = jnp.sum(bt_ref[...], axis=1, keepdims=True) * _LOG2E
    B = out_ref.shape[0]
    Z = g_ref.shape[0]
    nch = _TS_B // _CH_B

    def chunk(c, carry):
        sl = pl.ds(c * _CH_B, _CH_B)
        for b in range(B):
            acc = None
            for z0 in range(0, Z, _ZGRP):
                uprod = None
                for zz in range(z0, z0 + _ZGRP):
                    y = g_ref[zz, b, sl][None, :] + bs
                    yi = jax.lax.bitcast_convert_type(y, jnp.int32)
                    ny = jax.lax.bitcast_convert_type(
                        yi | jnp.int32(-2147483648), jnp.float32
                    )
                    u = 1.0 + jnp.exp2(ny)
                    uprod = u if uprod is None else uprod * u
                    m = jnp.maximum(y, 0.0)
                    acc = m if acc is None else acc + m
                acc = acc + jnp.log2(uprod)
            out_ref[b, :, sl] = acc * _LN2
        return carry

    lax.fori_loop(0, nch, chunk, 0, unroll=2)


def _stage_b(g, bt):
    Z, B, S = g.shape
    NSpec = bt.shape[0]
    nt = pl.cdiv(S, _TS_B)
    return pl.pallas_call(
        _stage_b_body,
        grid=(nt,),
        in_specs=[
            pl.BlockSpec((Z, B, _TS_B), lambda t: (0, 0, t)),
            pl.BlockSpec(bt.shape, lambda t: (0, 0)),
        ],
        out_specs=pl.BlockSpec((B, NSpec, _TS_B), lambda t: (0, 0, t)),
        out_shape=jax.ShapeDtypeStruct((B, NSpec, S), jnp.float32),
    )(g, bt)


def kernel(In, NNsites, Weights, bias):
    wt = Weights[:, 0, :].T
    bt = bias.T
    nn = NNsites[1:]
    p1, p2 = _stage_a(In, wt)
    g = _sc_gather(p1, p2, nn)
    return _stage_b(g, bt)
